# baseline (speedup 1.0000x reference)
import functools

import jax
import jax.numpy as jnp
from jax import lax
from jax.experimental import pallas as pl
from jax.experimental.pallas import tpu as pltpu
from jax.experimental.pallas import tpu_sc as plsc

EPS_ = 1e-07
MAX_NORM_ = 1 - 0.0001

NUM_WORKERS = 32
ROWS_PER_STEP = 128
D = 16


def _project_chunk(rows_v):
    lane = lax.iota(jnp.int32, 16)

    def block(b, carry):
        row_ids = lane + b * 16
        cols = []
        ssum = jnp.zeros((16,), jnp.float32)
        for d in range(D):
            col = plsc.load_gather(rows_v, [row_ids, jnp.full((16,), d, jnp.int32)])
            cols.append(col)
            ssum = ssum + col * col
        bits = lax.bitcast_convert_type(ssum, jnp.int32)
        y = lax.bitcast_convert_type(
            jnp.int32(0x5F3759DF) - (bits >> 1), jnp.float32)
        for _ in range(3):
            y = y * (1.5 - 0.5 * ssum * y * y)
        norm = ssum * y
        factor = jnp.where(norm >= MAX_NORM_, MAX_NORM_ * y,
                           jnp.ones((16,), jnp.float32))
        for d in range(D):
            plsc.store_scatter(rows_v, [row_ids, jnp.full((16,), d, jnp.int32)],
                               cols[d] * factor)
        return carry

    lax.fori_loop(0, ROWS_PER_STEP // 16, block, 0)


def _make_sc_kernel(n_flat):
    rows_per_worker = n_flat // NUM_WORKERS
    chunks = rows_per_worker // ROWS_PER_STEP
    info = plsc.get_sparse_core_info()
    nc = info.num_cores
    mesh = plsc.VectorSubcoreMesh(core_axis_name="c", subcore_axis_name="s")

    @functools.partial(
        pl.kernel,
        mesh=mesh,
        out_type=jax.ShapeDtypeStruct((n_flat, D), jnp.float32),
        compiler_params=pltpu.CompilerParams(needs_layout_passes=False,
                                             use_tc_tiling_on_sc=False),
        scratch_types=[
            pltpu.VMEM((chunks, ROWS_PER_STEP), jnp.int32),
            pltpu.VMEM((ROWS_PER_STEP, D), jnp.float32),
            pltpu.SemaphoreType.DMA,
        ],
    )
    def sc_kernel(idx_hbm, emb_hbm, out_hbm, idx_v, rows_v, sem):
        wid = lax.axis_index("s") * nc + lax.axis_index("c")
        pltpu.sync_copy(idx_hbm.at[wid], idx_v)
        base = wid * rows_per_worker

        def step(j, carry):
            pltpu.async_copy(emb_hbm.at[idx_v.at[j]], rows_v, sem).wait()
            _project_chunk(rows_v)
            pltpu.sync_copy(
                rows_v,
                out_hbm.at[pl.ds(base + j * ROWS_PER_STEP, ROWS_PER_STEP)])
            return carry

        lax.fori_loop(0, chunks, step, 0)

    return sc_kernel


def kernel(idx, emb):
    b, s = idx.shape
    n_flat = b * s
    idx_flat = idx.reshape(NUM_WORKERS,
                           n_flat // (NUM_WORKERS * ROWS_PER_STEP),
                           ROWS_PER_STEP).astype(jnp.int32)
    out = _make_sc_kernel(n_flat)(idx_flat, emb)
    return out.reshape(b, s, D)


# speedup vs baseline: 1.0557x; 1.0557x over previous
import functools

import jax
import jax.numpy as jnp
from jax import lax
from jax.experimental import pallas as pl
from jax.experimental.pallas import tpu as pltpu
from jax.experimental.pallas import tpu_sc as plsc

EPS_ = 1e-07
MAX_NORM_ = 1 - 0.0001

NUM_WORKERS = 32
ROWS_PER_STEP = 128
D = 16


def _project_chunk(rows_v):
    lane = lax.iota(jnp.int32, 16)

    def block(b, carry):
        row_ids = lane + b * 16
        cols = []
        ssum = jnp.zeros((16,), jnp.float32)
        for d in range(D):
            col = plsc.load_gather(rows_v, [row_ids, jnp.full((16,), d, jnp.int32)])
            cols.append(col)
            ssum = ssum + col * col
        bits = lax.bitcast_convert_type(ssum, jnp.int32)
        y = lax.bitcast_convert_type(
            jnp.int32(0x5F3759DF) - (bits >> 1), jnp.float32)
        for _ in range(3):
            y = y * (1.5 - 0.5 * ssum * y * y)
        norm = ssum * y
        factor = jnp.where(norm >= MAX_NORM_, MAX_NORM_ * y,
                           jnp.ones((16,), jnp.float32))
        for d in range(D):
            plsc.store_scatter(rows_v, [row_ids, jnp.full((16,), d, jnp.int32)],
                               cols[d] * factor)
        return carry

    lax.fori_loop(0, ROWS_PER_STEP // 16, block, 0)


def _make_sc_kernel(n_flat):
    rows_per_worker = n_flat // NUM_WORKERS
    chunks = rows_per_worker // ROWS_PER_STEP
    info = plsc.get_sparse_core_info()
    nc = info.num_cores
    mesh = plsc.VectorSubcoreMesh(core_axis_name="c", subcore_axis_name="s")

    @functools.partial(
        pl.kernel,
        mesh=mesh,
        out_type=jax.ShapeDtypeStruct((n_flat, D), jnp.float32),
        compiler_params=pltpu.CompilerParams(needs_layout_passes=False,
                                             use_tc_tiling_on_sc=False),
        scratch_types=[
            pltpu.VMEM((chunks, ROWS_PER_STEP), jnp.int32),
            pltpu.VMEM((ROWS_PER_STEP, D), jnp.float32),
            pltpu.VMEM((ROWS_PER_STEP, D), jnp.float32),
            pltpu.SemaphoreType.DMA,
            pltpu.SemaphoreType.DMA,
        ],
    )
    def sc_kernel(idx_hbm, emb_hbm, out_hbm, idx_v, rows_a, rows_b, sem_a,
                  sem_b):
        wid = lax.axis_index("s") * nc + lax.axis_index("c")
        pltpu.sync_copy(idx_hbm.at[wid], idx_v)
        base = wid * rows_per_worker

        pltpu.async_copy(emb_hbm.at[idx_v.at[0]], rows_a, sem_a)

        def halfstep(c, buf, sem, nbuf, nsem):
            pltpu.make_async_copy(emb_hbm.at[idx_v.at[c]], buf, sem).wait()

            @pl.when(c + 1 < chunks)
            def _():
                pltpu.async_copy(emb_hbm.at[idx_v.at[c + 1]], nbuf, nsem)

            _project_chunk(buf)
            pltpu.sync_copy(
                buf, out_hbm.at[pl.ds(base + c * ROWS_PER_STEP,
                                      ROWS_PER_STEP)])

        def step(t, carry):
            halfstep(2 * t, rows_a, sem_a, rows_b, sem_b)
            halfstep(2 * t + 1, rows_b, sem_b, rows_a, sem_a)
            return carry

        lax.fori_loop(0, chunks // 2, step, 0)

    return sc_kernel


def kernel(idx, emb):
    b, s = idx.shape
    n_flat = b * s
    idx_flat = idx.reshape(NUM_WORKERS,
                           n_flat // (NUM_WORKERS * ROWS_PER_STEP),
                           ROWS_PER_STEP).astype(jnp.int32)
    out = _make_sc_kernel(n_flat)(idx_flat, emb)
    return out.reshape(b, s, D)


# speedup vs baseline: 1.0561x; 1.0004x over previous
import functools

import jax
import jax.numpy as jnp
from jax import lax
from jax.experimental import pallas as pl
from jax.experimental.pallas import tpu as pltpu
from jax.experimental.pallas import tpu_sc as plsc

EPS_ = 1e-07
MAX_NORM_ = 1 - 0.0001

NUM_WORKERS = 32
ROWS_PER_STEP = 128
D = 16


def _project_chunk(rows_v):
    lane = lax.iota(jnp.int32, 16)

    def block(b, carry):
        row_ids = lane + b * 16
        diags = []
        ssum = jnp.zeros((16,), jnp.float32)
        for j in range(D):
            col_ids = (lane + j) & (D - 1)
            dg = plsc.load_gather(rows_v, [row_ids, col_ids])
            diags.append(dg)
            ssum = ssum + dg * dg
        bits = lax.bitcast_convert_type(ssum, jnp.int32)
        y = lax.bitcast_convert_type(
            jnp.int32(0x5F3759DF) - (bits >> 1), jnp.float32)
        for _ in range(3):
            y = y * (1.5 - 0.5 * ssum * y * y)
        norm = ssum * y
        factor = jnp.where(norm >= MAX_NORM_, MAX_NORM_ * y,
                           jnp.ones((16,), jnp.float32))
        for j in range(D):
            col_ids = (lane + j) & (D - 1)
            plsc.store_scatter(rows_v, [row_ids, col_ids], diags[j] * factor)
        return carry

    lax.fori_loop(0, ROWS_PER_STEP // 16, block, 0)


def _make_sc_kernel(n_flat):
    rows_per_worker = n_flat // NUM_WORKERS
    chunks = rows_per_worker // ROWS_PER_STEP
    info = plsc.get_sparse_core_info()
    nc = info.num_cores
    mesh = plsc.VectorSubcoreMesh(core_axis_name="c", subcore_axis_name="s")

    @functools.partial(
        pl.kernel,
        mesh=mesh,
        out_type=jax.ShapeDtypeStruct((n_flat, D), jnp.float32),
        compiler_params=pltpu.CompilerParams(needs_layout_passes=False,
                                             use_tc_tiling_on_sc=False),
        scratch_types=[
            pltpu.VMEM((chunks, ROWS_PER_STEP), jnp.int32),
            pltpu.VMEM((ROWS_PER_STEP, D), jnp.float32),
            pltpu.VMEM((ROWS_PER_STEP, D), jnp.float32),
            pltpu.SemaphoreType.DMA,
            pltpu.SemaphoreType.DMA,
            pltpu.SemaphoreType.DMA,
            pltpu.SemaphoreType.DMA,
        ],
    )
    def sc_kernel(idx_hbm, emb_hbm, out_hbm, idx_v, rows_a, rows_b, gsem_a,
                  gsem_b, ssem_a, ssem_b):
        wid = lax.axis_index("s") * nc + lax.axis_index("c")
        pltpu.sync_copy(idx_hbm.at[wid], idx_v)
        base = wid * rows_per_worker

        def out_at(c):
            return out_hbm.at[pl.ds(base + c * ROWS_PER_STEP, ROWS_PER_STEP)]

        pltpu.async_copy(emb_hbm.at[idx_v.at[0]], rows_a, gsem_a)

        def halfstep(c, buf, gsem, ssem, nbuf, ngsem, nssem):
            pltpu.make_async_copy(emb_hbm.at[idx_v.at[c]], buf, gsem).wait()

            @pl.when(c >= 1)
            def _():
                pltpu.make_async_copy(nbuf, out_at(c - 1), nssem).wait()

            @pl.when(c + 1 < chunks)
            def _():
                pltpu.async_copy(emb_hbm.at[idx_v.at[c + 1]], nbuf, ngsem)

            _project_chunk(buf)
            pltpu.async_copy(buf, out_at(c), ssem)

        def step(t, carry):
            halfstep(2 * t, rows_a, gsem_a, ssem_a, rows_b, gsem_b, ssem_b)
            halfstep(2 * t + 1, rows_b, gsem_b, ssem_b, rows_a, gsem_a,
                     ssem_a)
            return carry

        lax.fori_loop(0, chunks // 2, step, 0)
        pltpu.make_async_copy(rows_b, out_at(chunks - 1), ssem_b).wait()

    return sc_kernel


def kernel(idx, emb):
    b, s = idx.shape
    n_flat = b * s
    idx_flat = idx.reshape(NUM_WORKERS,
                           n_flat // (NUM_WORKERS * ROWS_PER_STEP),
                           ROWS_PER_STEP).astype(jnp.int32)
    out = _make_sc_kernel(n_flat)(idx_flat, emb)
    return out.reshape(b, s, D)


# speedup vs baseline: 1.3983x; 1.3240x over previous
import functools

import jax
import jax.numpy as jnp
from jax import lax
from jax.experimental import pallas as pl
from jax.experimental.pallas import tpu as pltpu
from jax.experimental.pallas import tpu_sc as plsc

EPS_ = 1e-07
MAX_NORM_ = 1 - 0.0001

NUM_WORKERS = 32
IDX_ROWS_PER_CHUNK = 8
D = 16


def _project_chunk(buf, n_rows, seq_len):
    lane = lax.iota(jnp.int32, 16)

    def block(b, carry):
        f = lane + b * 16
        d0 = f // seq_len
        d1 = f % seq_len
        diags = []
        ssum = jnp.zeros((16,), jnp.float32)
        for j in range(D):
            d2 = (lane + j) & (D - 1)
            dg = plsc.load_gather(buf, [d0, d1, d2])
            diags.append(dg)
            ssum = ssum + dg * dg
        bits = lax.bitcast_convert_type(ssum, jnp.int32)
        y = lax.bitcast_convert_type(
            jnp.int32(0x5F3759DF) - (bits >> 1), jnp.float32)
        for _ in range(3):
            y = y * (1.5 - 0.5 * ssum * y * y)
        norm = ssum * y
        factor = jnp.where(norm >= MAX_NORM_, MAX_NORM_ * y,
                           jnp.ones((16,), jnp.float32))
        for j in range(D):
            d2 = (lane + j) & (D - 1)
            plsc.store_scatter(buf, [d0, d1, d2], diags[j] * factor)
        return carry

    lax.fori_loop(0, n_rows // 16, block, 0)


def _make_sc_kernel(n_idx, seq_len):
    idx_rows_per_worker = n_idx // NUM_WORKERS
    chunks = idx_rows_per_worker // IDX_ROWS_PER_CHUNK
    rows_per_chunk = IDX_ROWS_PER_CHUNK * seq_len
    info = plsc.get_sparse_core_info()
    nc = info.num_cores
    mesh = plsc.VectorSubcoreMesh(core_axis_name="c", subcore_axis_name="s")
    buf_t = pltpu.VMEM((IDX_ROWS_PER_CHUNK, seq_len, D), jnp.float32)

    @functools.partial(
        pl.kernel,
        mesh=mesh,
        out_type=jax.ShapeDtypeStruct((n_idx, seq_len, D), jnp.float32),
        compiler_params=pltpu.CompilerParams(needs_layout_passes=False,
                                             use_tc_tiling_on_sc=False),
        scratch_types=[
            pltpu.VMEM((idx_rows_per_worker, seq_len), jnp.int32),
            buf_t,
            buf_t,
            pltpu.SemaphoreType.DMA,
            pltpu.SemaphoreType.DMA,
            pltpu.SemaphoreType.DMA,
            pltpu.SemaphoreType.DMA,
        ],
    )
    def sc_kernel(idx_hbm, emb_hbm, out_hbm, idx_v, rows_a, rows_b, gsem_a,
                  gsem_b, ssem_a, ssem_b):
        wid = lax.axis_index("s") * nc + lax.axis_index("c")
        base = wid * idx_rows_per_worker
        pltpu.sync_copy(idx_hbm.at[pl.ds(base, idx_rows_per_worker)], idx_v)

        def start_gathers(c, buf, gsem):
            r0 = c * IDX_ROWS_PER_CHUNK
            for k in range(IDX_ROWS_PER_CHUNK):
                pltpu.make_async_copy(
                    emb_hbm.at[idx_v.at[r0 + k]], buf.at[k], gsem).start()

        def wait_gathers(c, buf, gsem):
            r0 = c * IDX_ROWS_PER_CHUNK
            for k in range(IDX_ROWS_PER_CHUNK):
                pltpu.make_async_copy(
                    emb_hbm.at[idx_v.at[r0 + k]], buf.at[k], gsem).wait()

        def out_at(c):
            return out_hbm.at[pl.ds(base + c * IDX_ROWS_PER_CHUNK,
                                    IDX_ROWS_PER_CHUNK)]

        start_gathers(0, rows_a, gsem_a)

        def halfstep(c, buf, gsem, ssem, nbuf, ngsem, nssem):
            wait_gathers(c, buf, gsem)

            @pl.when(c >= 1)
            def _():
                pltpu.make_async_copy(nbuf, out_at(c - 1), nssem).wait()

            @pl.when(c + 1 < chunks)
            def _():
                start_gathers(c + 1, nbuf, ngsem)

            _project_chunk(buf, rows_per_chunk, seq_len)
            pltpu.make_async_copy(buf, out_at(c), ssem).start()

        def step(t, carry):
            halfstep(2 * t, rows_a, gsem_a, ssem_a, rows_b, gsem_b, ssem_b)
            halfstep(2 * t + 1, rows_b, gsem_b, ssem_b, rows_a, gsem_a,
                     ssem_a)
            return carry

        lax.fori_loop(0, chunks // 2, step, 0)
        pltpu.make_async_copy(rows_b, out_at(chunks - 1), ssem_b).wait()

    return sc_kernel


def kernel(idx, emb):
    n_idx, seq_len = idx.shape
    return _make_sc_kernel(n_idx, seq_len)(idx.astype(jnp.int32), emb)
